# Initial kernel scaffold; baseline (speedup 1.0000x reference)
import jax
import jax.numpy as jnp
from jax.experimental import pallas as pl


def kernel(x_paper, x_author, ei_cites, ei_writes, ei_rev, params):
    raise NotImplementedError("write your pallas kernel here")


# speedup vs baseline: 1.0021x; 1.0021x over previous
import jax
import jax.numpy as jnp
from jax.experimental import pallas as pl

HEADS = 8
HID = 128
NEG_SLOPE = 0.2
EDGE_TYPES = {'cites': ('paper', 'paper'), 'writes': ('author', 'paper'),
              'rev': ('paper', 'author')}


def _edge_softmax(alpha, dst, n):
    amax = jax.ops.segment_max(alpha, dst, num_segments=n)
    amax = jax.lax.stop_gradient(jnp.where(jnp.isfinite(amax), amax, 0.0))
    e = jnp.exp(alpha - amax[dst])
    s = jax.ops.segment_sum(e, dst, num_segments=n)
    return e / (s[dst] + 1e-16)


def _han_layer(x_dict, ei_dict, lp):
    xt = {}
    for nt, x in x_dict.items():
        W, b = lp['proj'][nt]
        xt[nt] = (x @ W + b).reshape(x.shape[0], HEADS, HID // HEADS)
    outs = {nt: [] for nt in x_dict}
    for ename in ['cites', 'writes', 'rev']:
        src_t, dst_t = EDGE_TYPES[ename]
        ei = ei_dict[ename]
        a_s, a_d = lp['att'][ename]
        x_s, x_d = xt[src_t], xt[dst_t]
        alpha_src = (x_s * a_s[None]).sum(-1)
        alpha_dst = (x_d * a_d[None]).sum(-1)
        j, i = ei[0], ei[1]
        alpha = jax.nn.leaky_relu(alpha_src[j] + alpha_dst[i], NEG_SLOPE)
        alpha = _edge_softmax(alpha, i, x_d.shape[0])
        msg = x_s[j] * alpha[:, :, None]
        agg = jax.ops.segment_sum(msg, i, num_segments=x_d.shape[0])
        outs[dst_t].append(jax.nn.relu(agg.reshape(agg.shape[0], -1)))
    kW, kb = lp['k']
    q = lp['q']
    res = {}
    for nt, xs in outs.items():
        st = jnp.stack(xs)
        score = (jnp.tanh(st @ kW + kb).mean(axis=1) * q[None]).sum(-1)
        attn = jax.nn.softmax(score)
        res[nt] = (attn[:, None, None] * st).sum(0)
    return res


def _linear_kernel(x_ref, w_ref, b_ref, o_ref):
    o_ref[...] = jnp.dot(x_ref[...], w_ref[...],
                         preferred_element_type=jnp.float32) + b_ref[...]


def _pallas_linear(x, W, b):
    n = x.shape[0]
    bn = 1024
    pad = (-n) % bn
    xp = jnp.pad(x, ((0, pad), (0, 0)))
    out = pl.pallas_call(
        _linear_kernel,
        grid=((n + pad) // bn,),
        in_specs=[pl.BlockSpec((bn, HID), lambda i: (i, 0)),
                  pl.BlockSpec((HID, W.shape[1]), lambda i: (0, 0)),
                  pl.BlockSpec((W.shape[1],), lambda i: (0,))],
        out_specs=pl.BlockSpec((bn, W.shape[1]), lambda i: (i, 0)),
        out_shape=jax.ShapeDtypeStruct((n + pad, W.shape[1]), jnp.float32),
    )(xp, W, b)
    return out[:n]


def kernel(x_paper, x_author, ei_cites, ei_writes, ei_rev, params):
    x_dict = {'paper': x_paper, 'author': x_author}
    ei_dict = {'cites': ei_cites, 'writes': ei_writes, 'rev': ei_rev}
    h = _han_layer(x_dict, ei_dict, params['l1'])
    h = {nt: jax.nn.elu(v) for nt, v in h.items()}
    h = _han_layer(h, ei_dict, params['l2'])
    h = {nt: jax.nn.elu(v) for nt, v in h.items()}
    W, b = params['lin']
    return (_pallas_linear(h['paper'], W, b), _pallas_linear(h['author'], W, b))


# speedup vs baseline: 32.5420x; 32.4732x over previous
import jax
import jax.numpy as jnp
from jax import lax
from jax.experimental import pallas as pl
from jax.experimental.pallas import tpu as pltpu
from jax.experimental.pallas import tpu_sc as plsc

HEADS = 8
HID = 128
GH = 16
NG = 4
GW = 32
NEG = 0.2
NSUB = 16
CHUNK = 128
EALIGN = NSUB * CHUNK
BN = 1024
EPS = 1e-16


def _rup(n, m):
    return ((n + m - 1) // m) * m


def _proj_tail(y, a_ref, outs, ntabs):
    xg = outs[:NG]
    ts = outs[NG:]
    for g in range(NG):
        xg[g][...] = y[:, g * GW:(g + 1) * GW]
    t = jnp.dot(y, a_ref[...], preferred_element_type=jnp.float32)
    for k in range(ntabs):
        ts[k][...] = t[:, k * 16:(k + 1) * 16]


def _pre_raw_body(ntabs, x_ref, w_ref, b_ref, a_ref, *outs):
    y = jnp.dot(x_ref[...], w_ref[...],
                preferred_element_type=jnp.float32) + b_ref[...]
    _proj_tail(y, a_ref, outs, ntabs)


def _elu(x):
    return jnp.where(x > 0, x, jnp.exp(jnp.minimum(x, 0.0)) - 1.0)


def _assemble(a_refs, s_ref):
    cat = jnp.concatenate([a[0] for a in a_refs], axis=-1)
    s = s_ref[...]
    cols = []
    for h in range(HEADS):
        denom = s[:, h:h + 1] + EPS
        cols.append(cat[:, GH * h:GH * (h + 1)] / denom)
    return jnp.maximum(jnp.concatenate(cols, axis=-1), 0.0)


def _pre_mix_body(ntabs, oc_ref, ow_ref, attn_ref, w_ref, b_ref, a_ref, *outs):
    x = _elu(attn_ref[0] * oc_ref[...] + attn_ref[1] * ow_ref[...])
    y = jnp.dot(x, w_ref[...], preferred_element_type=jnp.float32) + b_ref[...]
    _proj_tail(y, a_ref, outs, ntabs)


def _pre_agg_body(ntabs, a0, a1, a2, a3, s_ref, w_ref, b_ref, a_ref, *outs):
    x = _elu(_assemble((a0, a1, a2, a3), s_ref))
    y = jnp.dot(x, w_ref[...], preferred_element_type=jnp.float32) + b_ref[...]
    _proj_tail(y, a_ref, outs, ntabs)


def _post_p_body(nvalid, c0, c1, c2, c3, sc_ref, w0, w1, w2, w3, sw_ref,
                 kw_ref, kb_ref, oc_ref, ow_ref, ts_ref):
    i = pl.program_id(0)
    oc = _assemble((c0, c1, c2, c3), sc_ref)
    ow = _assemble((w0, w1, w2, w3), sw_ref)
    oc_ref[...] = oc
    ow_ref[...] = ow
    tc = jnp.tanh(jnp.dot(oc, kw_ref[...],
                          preferred_element_type=jnp.float32) + kb_ref[...])
    tw = jnp.tanh(jnp.dot(ow, kw_ref[...],
                          preferred_element_type=jnp.float32) + kb_ref[...])
    ridx = i * BN + lax.broadcasted_iota(jnp.int32, (BN, 1), 0)
    m = (ridx < nvalid).astype(jnp.float32)
    part = jnp.stack([(tc * m).sum(0), (tw * m).sum(0)])

    @pl.when(i == 0)
    def _():
        ts_ref[...] = part

    @pl.when(i > 0)
    def _():
        ts_ref[...] = ts_ref[...] + part


def _final_mix_body(oc_ref, ow_ref, attn_ref, w_ref, b_ref, o_ref):
    x = _elu(attn_ref[0] * oc_ref[...] + attn_ref[1] * ow_ref[...])
    o_ref[...] = jnp.dot(x, w_ref[...],
                         preferred_element_type=jnp.float32) + b_ref[...]


def _final_agg_body(a0, a1, a2, a3, s_ref, w_ref, b_ref, o_ref):
    x = _elu(_assemble((a0, a1, a2, a3), s_ref))
    o_ref[...] = jnp.dot(x, w_ref[...],
                         preferred_element_type=jnp.float32) + b_ref[...]


def _mat_spec(shape):
    nd = len(shape)
    return pl.BlockSpec(shape, lambda i, _nd=nd: (0,) * _nd)


def _agg_view_specs():
    return [pl.BlockSpec((1, BN, GW), lambda i, g=g: (g, i, 0))
            for g in range(NG)]


def _pre_outs(npad, ntabs):
    shapes = ([jax.ShapeDtypeStruct((npad, GW), jnp.float32)] * NG
              + [jax.ShapeDtypeStruct((npad, 16), jnp.float32)] * ntabs)
    specs = ([pl.BlockSpec((BN, GW), lambda i: (i, 0))] * NG
             + [pl.BlockSpec((BN, 16), lambda i: (i, 0))] * ntabs)
    return shapes, specs


def _pre_raw(x, W, b, A, ntabs):
    npad = x.shape[0]
    oshapes, ospecs = _pre_outs(npad, ntabs)
    import functools
    return pl.pallas_call(
        functools.partial(_pre_raw_body, ntabs),
        grid=(npad // BN,),
        in_specs=[pl.BlockSpec((BN, HID), lambda i: (i, 0)),
                  _mat_spec((HID, HID)), _mat_spec((1, HID)),
                  _mat_spec((HID, 16 * ntabs))],
        out_specs=ospecs, out_shape=oshapes,
    )(x, W, b.reshape(1, HID), A)


def _pre_mix(oc, ow, attn, W, b, A, ntabs):
    npad = oc.shape[0]
    oshapes, ospecs = _pre_outs(npad, ntabs)
    import functools
    return pl.pallas_call(
        functools.partial(_pre_mix_body, ntabs),
        grid=(npad // BN,),
        in_specs=[pl.BlockSpec((BN, HID), lambda i: (i, 0)),
                  pl.BlockSpec((BN, HID), lambda i: (i, 0)),
                  pl.BlockSpec(memory_space=pltpu.SMEM),
                  _mat_spec((HID, HID)), _mat_spec((1, HID)),
                  _mat_spec((HID, 16 * ntabs))],
        out_specs=ospecs, out_shape=oshapes,
    )(oc, ow, attn, W, b.reshape(1, HID), A)


def _pre_agg(agg, s, W, b, A, ntabs):
    npad = s.shape[0]
    oshapes, ospecs = _pre_outs(npad, ntabs)
    import functools
    return pl.pallas_call(
        functools.partial(_pre_agg_body, ntabs),
        grid=(npad // BN,),
        in_specs=_agg_view_specs()
        + [pl.BlockSpec((BN, 16), lambda i: (i, 0)),
           _mat_spec((HID, HID)), _mat_spec((1, HID)),
           _mat_spec((HID, 16 * ntabs))],
        out_specs=ospecs, out_shape=oshapes,
    )(agg, agg, agg, agg, s, W, b.reshape(1, HID), A)


def _post_p(agg_c, s_c, agg_w, s_w, kW, kb, nvalid):
    npad = s_c.shape[0]
    import functools
    return pl.pallas_call(
        functools.partial(_post_p_body, nvalid),
        grid=(npad // BN,),
        in_specs=_agg_view_specs()
        + [pl.BlockSpec((BN, 16), lambda i: (i, 0))]
        + _agg_view_specs()
        + [pl.BlockSpec((BN, 16), lambda i: (i, 0)),
           _mat_spec((HID, HID)), _mat_spec((1, HID))],
        out_specs=[pl.BlockSpec((BN, HID), lambda i: (i, 0)),
                   pl.BlockSpec((BN, HID), lambda i: (i, 0)),
                   pl.BlockSpec((2, HID), lambda i: (0, 0))],
        out_shape=[jax.ShapeDtypeStruct((npad, HID), jnp.float32),
                   jax.ShapeDtypeStruct((npad, HID), jnp.float32),
                   jax.ShapeDtypeStruct((2, HID), jnp.float32)],
    )(agg_c, agg_c, agg_c, agg_c, s_c, agg_w, agg_w, agg_w, agg_w, s_w,
      kW, kb.reshape(1, HID))


def _final_mix(oc, ow, attn, W, b):
    npad = oc.shape[0]
    return pl.pallas_call(
        _final_mix_body,
        grid=(npad // BN,),
        in_specs=[pl.BlockSpec((BN, HID), lambda i: (i, 0)),
                  pl.BlockSpec((BN, HID), lambda i: (i, 0)),
                  pl.BlockSpec(memory_space=pltpu.SMEM),
                  _mat_spec((HID, HID)), _mat_spec((1, HID))],
        out_specs=pl.BlockSpec((BN, HID), lambda i: (i, 0)),
        out_shape=jax.ShapeDtypeStruct((npad, HID), jnp.float32),
    )(oc, ow, attn, W, b.reshape(1, HID))


def _final_agg(agg, s, W, b):
    npad = s.shape[0]
    return pl.pallas_call(
        _final_agg_body,
        grid=(npad // BN,),
        in_specs=_agg_view_specs()
        + [pl.BlockSpec((BN, 16), lambda i: (i, 0)),
           _mat_spec((HID, HID)), _mat_spec((1, HID))],
        out_specs=pl.BlockSpec((BN, HID), lambda i: (i, 0)),
        out_shape=jax.ShapeDtypeStruct((npad, HID), jnp.float32),
    )(agg, agg, agg, agg, s, W, b.reshape(1, HID))


def _sc_mesh():
    return plsc.VectorSubcoreMesh(core_axis_name="c", subcore_axis_name="s")


def _sc_params():
    return pltpu.CompilerParams(use_tc_tiling_on_sc=False)


def _sc_edge_weights(NPP, NPA, EPc, EPw, EPr):
    f32 = jnp.float32

    def body(jc, ic, jw, iw, jr, ir, Sc, Dc, Sw, Dw, Sr, Dr,
             btc, btw, btr, z16,
             e_c, e_w, e_r, s_c, s_w, s_r,
             jv, iv, sv, dv, ev, btv, shc, shw, shr):
        c = lax.axis_index("c")
        sub = lax.axis_index("s")

        def zero_tab(sh, nrows):
            rows = nrows // NSUB
            r0 = sub * rows
            pltpu.sync_copy(z16.at[pl.ds(0, rows)], sh.at[pl.ds(r0, rows)])

        @pl.when(c == 0)
        def _():
            zero_tab(shc, NPP)

        @pl.when(c == 1)
        def _():
            zero_tab(shw, NPP)
            zero_tab(shr, NPA)

        plsc.subcore_barrier()

        def run_edges(j_h, i_h, S_h, D_h, bt_h, e_h, sh, EP):
            pltpu.sync_copy(bt_h, btv)
            nper = EP // NSUB
            nch = nper // CHUNK

            @pl.loop(0, nch)
            def _(k):
                base = sub * nper + k * CHUNK
                pltpu.sync_copy(j_h.at[pl.ds(base, CHUNK)], jv)
                pltpu.sync_copy(i_h.at[pl.ds(base, CHUNK)], iv)
                pltpu.sync_copy(S_h.at[jv], sv)
                pltpu.sync_copy(D_h.at[iv], dv)
                bt = btv[...]

                @pl.loop(0, CHUNK)
                def _(r):
                    a = (sv.at[pl.ds(r, 1), pl.ds(0, 16)][...]
                         + dv.at[pl.ds(r, 1), pl.ds(0, 16)][...])
                    al = jnp.maximum(a, NEG * a)
                    ev.at[pl.ds(r, 1), pl.ds(0, 16)][...] = jnp.exp(al - bt)

                pltpu.sync_copy(ev, e_h.at[pl.ds(base, CHUNK)])
                pltpu.sync_copy(ev, sh.at[iv], add=True)

        @pl.when(c == 0)
        def _():
            run_edges(jc, ic, Sc, Dc, btc, e_c, shc, EPc)

        @pl.when(c == 1)
        def _():
            run_edges(jw, iw, Sw, Dw, btw, e_w, shw, EPw)
            run_edges(jr, ir, Sr, Dr, btr, e_r, shr, EPr)

        plsc.subcore_barrier()

        def flush(sh, out, nrows):
            rows = nrows // NSUB
            r0 = sub * rows
            pltpu.sync_copy(sh.at[pl.ds(r0, rows)], out.at[pl.ds(r0, rows)])

        @pl.when(c == 0)
        def _():
            flush(shc, s_c, NPP)

        @pl.when(c == 1)
        def _():
            flush(shw, s_w, NPP)
            flush(shr, s_r, NPA)

    return pl.kernel(
        body,
        mesh=_sc_mesh(),
        compiler_params=_sc_params(),
        out_type=[jax.ShapeDtypeStruct((EPc, 16), f32),
                  jax.ShapeDtypeStruct((EPw, 16), f32),
                  jax.ShapeDtypeStruct((EPr, 16), f32),
                  jax.ShapeDtypeStruct((NPP, 16), f32),
                  jax.ShapeDtypeStruct((NPP, 16), f32),
                  jax.ShapeDtypeStruct((NPA, 16), f32)],
        scratch_types=[pltpu.VMEM((CHUNK,), jnp.int32),
                       pltpu.VMEM((CHUNK,), jnp.int32),
                       pltpu.VMEM((CHUNK, 16), f32),
                       pltpu.VMEM((CHUNK, 16), f32),
                       pltpu.VMEM((CHUNK, 16), f32),
                       pltpu.VMEM((16,), f32),
                       pltpu.VMEM_SHARED((NPP, 16), f32),
                       pltpu.VMEM_SHARED((NPP, 16), f32),
                       pltpu.VMEM_SHARED((NPA, 16), f32)],
    )


def _sc_aggregate(NPP, NPA, EPc, EPw, EPr):
    f32 = jnp.float32

    def body(jc, ic, jw, iw, jr, ir, ec, ew, er,
             xp0, xp1, xp2, xp3, xa0, xa1, xa2, xa3, z32,
             agg_c, agg_w, agg_r,
             jv, iv, xv, ev, mv, sh):
        c = lax.axis_index("c")
        sub = lax.axis_index("s")
        xps = (xp0, xp1, xp2, xp3)
        xas = (xa0, xa1, xa2, xa3)

        def zero_round(nrows):
            rows = nrows // NSUB
            r0 = sub * rows
            pltpu.sync_copy(z32.at[pl.ds(0, rows)], sh.at[pl.ds(r0, rows)])

        def acc_round(j_h, i_h, e_h, xg_h, g, EP):
            nper = EP // NSUB
            nch = nper // CHUNK

            @pl.loop(0, nch)
            def _(k):
                base = sub * nper + k * CHUNK
                pltpu.sync_copy(j_h.at[pl.ds(base, CHUNK)], jv)
                pltpu.sync_copy(i_h.at[pl.ds(base, CHUNK)], iv)
                pltpu.sync_copy(e_h.at[pl.ds(base, CHUNK)], ev)
                pltpu.sync_copy(xg_h.at[jv], xv)

                @pl.loop(0, CHUNK)
                def _(r):
                    er = ev.at[pl.ds(r, 1), pl.ds(0, 16)][...]
                    e0 = er[0, 2 * g]
                    e1 = er[0, 2 * g + 1]
                    x0 = xv.at[pl.ds(r, 1), pl.ds(0, 16)][...]
                    x1 = xv.at[pl.ds(r, 1), pl.ds(16, 16)][...]
                    mv.at[pl.ds(r, 1), pl.ds(0, 16)][...] = x0 * e0
                    mv.at[pl.ds(r, 1), pl.ds(16, 16)][...] = x1 * e1

                pltpu.sync_copy(mv, sh.at[iv], add=True)

        def flush_round(agg_out, g, nrows):
            rows = nrows // NSUB
            r0 = sub * rows
            pltpu.sync_copy(sh.at[pl.ds(r0, rows)],
                            agg_out.at[g, pl.ds(r0, rows)])

        for r in range(8):
            g = r % 4
            if r < 4:
                @pl.when(c == 0)
                def _():
                    zero_round(NPP)

            @pl.when(c == 1)
            def _():
                zero_round(NPP if r < 4 else NPA)

            plsc.subcore_barrier()

            if r < 4:
                @pl.when(c == 0)
                def _():
                    acc_round(jc, ic, ec, xps[g], g, EPc)

            @pl.when(c == 1)
            def _():
                if r < 4:
                    acc_round(jw, iw, ew, xas[g], g, EPw)
                else:
                    acc_round(jr, ir, er, xps[g], g, EPr)

            plsc.subcore_barrier()

            if r < 4:
                @pl.when(c == 0)
                def _():
                    flush_round(agg_c, g, NPP)

            @pl.when(c == 1)
            def _():
                if r < 4:
                    flush_round(agg_w, g, NPP)
                else:
                    flush_round(agg_r, g, NPA)

            plsc.subcore_barrier()

    return pl.kernel(
        body,
        mesh=_sc_mesh(),
        compiler_params=_sc_params(),
        out_type=[jax.ShapeDtypeStruct((NG, NPP, GW), f32),
                  jax.ShapeDtypeStruct((NG, NPP, GW), f32),
                  jax.ShapeDtypeStruct((NG, NPA, GW), f32)],
        scratch_types=[pltpu.VMEM((CHUNK,), jnp.int32),
                       pltpu.VMEM((CHUNK,), jnp.int32),
                       pltpu.VMEM((CHUNK, GW), f32),
                       pltpu.VMEM((CHUNK, 16), f32),
                       pltpu.VMEM((CHUNK, GW), f32),
                       pltpu.VMEM_SHARED((NPP, GW), f32)],
    )


def _att_mat(a):
    m = jnp.einsum('hd,hk->hdk', a, jnp.eye(HEADS, dtype=a.dtype))
    return jnp.pad(m.reshape(HID, HEADS), ((0, 0), (0, 8)))


def _bound(Ts, Td):
    raw = Ts.max(0) + Td.max(0)
    return jnp.maximum(raw, NEG * raw)


def kernel(x_paper, x_author, ei_cites, ei_writes, ei_rev, params):
    NPv, NAv = x_paper.shape[0], x_author.shape[0]
    NPP = _rup(NPv + 1, BN)
    NPA = _rup(NAv + 1, BN)
    Ec, Ew, Er = ei_cites.shape[1], ei_writes.shape[1], ei_rev.shape[1]
    EPc, EPw, EPr = _rup(Ec, EALIGN), _rup(Ew, EALIGN), _rup(Er, EALIGN)

    xp = jnp.pad(x_paper, ((0, NPP - NPv), (0, 0)))
    xa = jnp.pad(x_author, ((0, NPA - NAv), (0, 0)))

    def pad_ei(ei, EP, dsrc, ddst):
        j = jnp.pad(ei[0], (0, EP - ei.shape[1]), constant_values=dsrc)
        i = jnp.pad(ei[1], (0, EP - ei.shape[1]), constant_values=ddst)
        return j, i

    jc, ic = pad_ei(ei_cites, EPc, NPv, NPv)
    jw, iw = pad_ei(ei_writes, EPw, NAv, NPv)
    jr, ir = pad_ei(ei_rev, EPr, NPv, NAv)

    z16 = jnp.zeros((NPP, 16), jnp.float32)
    z32 = jnp.zeros((NPP, GW), jnp.float32)

    kA = _sc_edge_weights(NPP, NPA, EPc, EPw, EPr)
    kB = _sc_aggregate(NPP, NPA, EPc, EPw, EPr)

    state = None
    for li in ('l1', 'l2'):
        lp = params[li]
        Wp, bp = lp['proj']['paper']
        Wa, ba = lp['proj']['author']
        asc, adc = lp['att']['cites']
        asw, adw = lp['att']['writes']
        asr, adr = lp['att']['rev']
        Ap = jnp.concatenate([_att_mat(asc), _att_mat(adc),
                              _att_mat(adw), _att_mat(asr)], axis=1)
        Aa = jnp.concatenate([_att_mat(asw), _att_mat(adr)], axis=1)

        if state is None:
            pp = _pre_raw(xp, Wp, bp, Ap, 4)
            pa = _pre_raw(xa, Wa, ba, Aa, 2)
        else:
            oc, ow, attn, agg_r_prev, s_r_prev = state
            pp = _pre_mix(oc, ow, attn, Wp, bp, Ap, 4)
            pa = _pre_agg(agg_r_prev, s_r_prev, Wa, ba, Aa, 2)
        xg_p = pp[:NG]
        Sc, Dc, Dw, Sr = pp[NG:]
        xg_a = pa[:NG]
        Sw, Dr = pa[NG:]

        btc = _bound(Sc, Dc)
        btw = _bound(Sw, Dw)
        btr = _bound(Sr, Dr)

        e_c, e_w, e_r, s_c, s_w, s_r = kA(
            jc, ic, jw, iw, jr, ir, Sc, Dc, Sw, Dw, Sr, Dr,
            btc, btw, btr, z16)
        agg_c, agg_w, agg_r = kB(
            jc, ic, jw, iw, jr, ir, e_c, e_w, e_r,
            *xg_p, *xg_a, z32)

        kW, kb = lp['k']
        q = lp['q']
        oc, ow, tsum = _post_p(agg_c, s_c, agg_w, s_w, kW, kb, NPv)
        score = ((tsum / NPv) * q[None, :]).sum(-1)
        attn = jax.nn.softmax(score)
        state = (oc, ow, attn, agg_r, s_r)

    oc, ow, attn, agg_r, s_r = state
    W, b = params['lin']
    out_p = _final_mix(oc, ow, attn, W, b)[:NPv]
    out_a = _final_agg(agg_r, s_r, W, b)[:NAv]
    return (out_p, out_a)


# speedup vs baseline: 45.6345x; 1.4023x over previous
import jax
import jax.numpy as jnp
from jax import lax
from jax.experimental import pallas as pl
from jax.experimental.pallas import tpu as pltpu
from jax.experimental.pallas import tpu_sc as plsc

HEADS = 8
HID = 128
GH = 16
NG = 4
GW = 32
NEG = 0.2
NSUB = 16
CHUNK = 128
SCHUNK = 256
EALIGN = NSUB * 512
BN = 1024
EPS = 1e-16


def _rup(n, m):
    return ((n + m - 1) // m) * m


def _proj_tail(y, a_ref, outs, ntabs):
    xg = outs[:NG]
    ts = outs[NG:]
    for g in range(NG):
        xg[g][...] = y[:, g * GW:(g + 1) * GW]
    t = jnp.dot(y, a_ref[...], preferred_element_type=jnp.float32)
    for k in range(ntabs):
        ts[k][...] = t[:, k * 16:(k + 1) * 16]


def _pre_raw_body(ntabs, x_ref, w_ref, b_ref, a_ref, *outs):
    y = jnp.dot(x_ref[...], w_ref[...],
                preferred_element_type=jnp.float32) + b_ref[...]
    _proj_tail(y, a_ref, outs, ntabs)


def _elu(x):
    return jnp.where(x > 0, x, jnp.exp(jnp.minimum(x, 0.0)) - 1.0)


def _assemble(a_refs, s_ref):
    cat = jnp.concatenate([a[0] for a in a_refs], axis=-1)
    s = s_ref[...]
    cols = []
    for h in range(HEADS):
        denom = s[:, h:h + 1] + EPS
        cols.append(cat[:, GH * h:GH * (h + 1)] / denom)
    return jnp.maximum(jnp.concatenate(cols, axis=-1), 0.0)


def _pre_mix_body(ntabs, oc_ref, ow_ref, attn_ref, w_ref, b_ref, a_ref, *outs):
    x = _elu(attn_ref[0] * oc_ref[...] + attn_ref[1] * ow_ref[...])
    y = jnp.dot(x, w_ref[...], preferred_element_type=jnp.float32) + b_ref[...]
    _proj_tail(y, a_ref, outs, ntabs)


def _pre_agg_body(ntabs, a0, a1, a2, a3, s_ref, w_ref, b_ref, a_ref, *outs):
    x = _elu(_assemble((a0, a1, a2, a3), s_ref))
    y = jnp.dot(x, w_ref[...], preferred_element_type=jnp.float32) + b_ref[...]
    _proj_tail(y, a_ref, outs, ntabs)


def _post_p_body(nvalid, c0, c1, c2, c3, sc_ref, w0, w1, w2, w3, sw_ref,
                 kw_ref, kb_ref, oc_ref, ow_ref, ts_ref):
    i = pl.program_id(0)
    oc = _assemble((c0, c1, c2, c3), sc_ref)
    ow = _assemble((w0, w1, w2, w3), sw_ref)
    oc_ref[...] = oc
    ow_ref[...] = ow
    tc = jnp.tanh(jnp.dot(oc, kw_ref[...],
                          preferred_element_type=jnp.float32) + kb_ref[...])
    tw = jnp.tanh(jnp.dot(ow, kw_ref[...],
                          preferred_element_type=jnp.float32) + kb_ref[...])
    ridx = i * BN + lax.broadcasted_iota(jnp.int32, (BN, 1), 0)
    m = (ridx < nvalid).astype(jnp.float32)
    part = jnp.stack([(tc * m).sum(0), (tw * m).sum(0)])

    @pl.when(i == 0)
    def _():
        ts_ref[...] = part

    @pl.when(i > 0)
    def _():
        ts_ref[...] = ts_ref[...] + part


def _final_mix_body(oc_ref, ow_ref, attn_ref, w_ref, b_ref, o_ref):
    x = _elu(attn_ref[0] * oc_ref[...] + attn_ref[1] * ow_ref[...])
    o_ref[...] = jnp.dot(x, w_ref[...],
                         preferred_element_type=jnp.float32) + b_ref[...]


def _final_agg_body(a0, a1, a2, a3, s_ref, w_ref, b_ref, o_ref):
    x = _elu(_assemble((a0, a1, a2, a3), s_ref))
    o_ref[...] = jnp.dot(x, w_ref[...],
                         preferred_element_type=jnp.float32) + b_ref[...]


def _mat_spec(shape):
    nd = len(shape)
    return pl.BlockSpec(shape, lambda i, _nd=nd: (0,) * _nd)


def _agg_view_specs():
    return [pl.BlockSpec((1, BN, GW), lambda i, g=g: (g, i, 0))
            for g in range(NG)]


def _pre_outs(npad, ntabs):
    shapes = ([jax.ShapeDtypeStruct((npad, GW), jnp.float32)] * NG
              + [jax.ShapeDtypeStruct((npad, 16), jnp.float32)] * ntabs)
    specs = ([pl.BlockSpec((BN, GW), lambda i: (i, 0))] * NG
             + [pl.BlockSpec((BN, 16), lambda i: (i, 0))] * ntabs)
    return shapes, specs


def _pre_raw(x, W, b, A, ntabs):
    npad = x.shape[0]
    oshapes, ospecs = _pre_outs(npad, ntabs)
    import functools
    return pl.pallas_call(
        functools.partial(_pre_raw_body, ntabs),
        grid=(npad // BN,),
        in_specs=[pl.BlockSpec((BN, HID), lambda i: (i, 0)),
                  _mat_spec((HID, HID)), _mat_spec((1, HID)),
                  _mat_spec((HID, 16 * ntabs))],
        out_specs=ospecs, out_shape=oshapes,
    )(x, W, b.reshape(1, HID), A)


def _pre_mix(oc, ow, attn, W, b, A, ntabs):
    npad = oc.shape[0]
    oshapes, ospecs = _pre_outs(npad, ntabs)
    import functools
    return pl.pallas_call(
        functools.partial(_pre_mix_body, ntabs),
        grid=(npad // BN,),
        in_specs=[pl.BlockSpec((BN, HID), lambda i: (i, 0)),
                  pl.BlockSpec((BN, HID), lambda i: (i, 0)),
                  pl.BlockSpec(memory_space=pltpu.SMEM),
                  _mat_spec((HID, HID)), _mat_spec((1, HID)),
                  _mat_spec((HID, 16 * ntabs))],
        out_specs=ospecs, out_shape=oshapes,
    )(oc, ow, attn, W, b.reshape(1, HID), A)


def _pre_agg(agg, s, W, b, A, ntabs):
    npad = s.shape[0]
    oshapes, ospecs = _pre_outs(npad, ntabs)
    import functools
    return pl.pallas_call(
        functools.partial(_pre_agg_body, ntabs),
        grid=(npad // BN,),
        in_specs=_agg_view_specs()
        + [pl.BlockSpec((BN, 16), lambda i: (i, 0)),
           _mat_spec((HID, HID)), _mat_spec((1, HID)),
           _mat_spec((HID, 16 * ntabs))],
        out_specs=ospecs, out_shape=oshapes,
    )(agg, agg, agg, agg, s, W, b.reshape(1, HID), A)


def _post_p(agg_c, s_c, agg_w, s_w, kW, kb, nvalid):
    npad = s_c.shape[0]
    import functools
    return pl.pallas_call(
        functools.partial(_post_p_body, nvalid),
        grid=(npad // BN,),
        in_specs=_agg_view_specs()
        + [pl.BlockSpec((BN, 16), lambda i: (i, 0))]
        + _agg_view_specs()
        + [pl.BlockSpec((BN, 16), lambda i: (i, 0)),
           _mat_spec((HID, HID)), _mat_spec((1, HID))],
        out_specs=[pl.BlockSpec((BN, HID), lambda i: (i, 0)),
                   pl.BlockSpec((BN, HID), lambda i: (i, 0)),
                   pl.BlockSpec((2, HID), lambda i: (0, 0))],
        out_shape=[jax.ShapeDtypeStruct((npad, HID), jnp.float32),
                   jax.ShapeDtypeStruct((npad, HID), jnp.float32),
                   jax.ShapeDtypeStruct((2, HID), jnp.float32)],
    )(agg_c, agg_c, agg_c, agg_c, s_c, agg_w, agg_w, agg_w, agg_w, s_w,
      kW, kb.reshape(1, HID))


def _final_mix(oc, ow, attn, W, b):
    npad = oc.shape[0]
    return pl.pallas_call(
        _final_mix_body,
        grid=(npad // BN,),
        in_specs=[pl.BlockSpec((BN, HID), lambda i: (i, 0)),
                  pl.BlockSpec((BN, HID), lambda i: (i, 0)),
                  pl.BlockSpec(memory_space=pltpu.SMEM),
                  _mat_spec((HID, HID)), _mat_spec((1, HID))],
        out_specs=pl.BlockSpec((BN, HID), lambda i: (i, 0)),
        out_shape=jax.ShapeDtypeStruct((npad, HID), jnp.float32),
    )(oc, ow, attn, W, b.reshape(1, HID))


def _final_agg(agg, s, W, b):
    npad = s.shape[0]
    return pl.pallas_call(
        _final_agg_body,
        grid=(npad // BN,),
        in_specs=_agg_view_specs()
        + [pl.BlockSpec((BN, 16), lambda i: (i, 0)),
           _mat_spec((HID, HID)), _mat_spec((1, HID))],
        out_specs=pl.BlockSpec((BN, HID), lambda i: (i, 0)),
        out_shape=jax.ShapeDtypeStruct((npad, HID), jnp.float32),
    )(agg, agg, agg, agg, s, W, b.reshape(1, HID))


def _sc_mesh():
    return plsc.VectorSubcoreMesh(core_axis_name="c", subcore_axis_name="s")


def _sc_params():
    return pltpu.CompilerParams(use_tc_tiling_on_sc=False)


def _sc_edge_weights(NPP, NPA, EPc, EPw, EPr):
    f32 = jnp.float32

    def body(jc, ic, jw, iw, jr, ir, Sc, Dc, Sw, Dw, Sr, Dr,
             btc, btw, btr, z16,
             e_c, e_w, e_r, s_c, s_w, s_r,
             jv, iv, sv, dv, ev, btv, shc, shw, shr):
        c = lax.axis_index("c")
        sub = lax.axis_index("s")

        def zero_tab(sh, nrows):
            rows = nrows // NSUB
            r0 = sub * rows
            pltpu.sync_copy(z16.at[pl.ds(0, rows)], sh.at[pl.ds(r0, rows)])

        @pl.when(c == 0)
        def _():
            zero_tab(shc, NPP)

        @pl.when(c == 1)
        def _():
            zero_tab(shw, NPP)
            zero_tab(shr, NPA)

        plsc.subcore_barrier()

        def run_edges(j_h, i_h, S_h, D_h, bt_h, e_h, sh, EP):
            pltpu.sync_copy(bt_h, btv)
            nch = EP // NSUB // CHUNK

            @pl.loop(0, nch)
            def _(k):
                row = sub * nch + k
                pltpu.sync_copy(j_h.at[pl.ds(row, 1)], jv)
                pltpu.sync_copy(i_h.at[pl.ds(row, 1)], iv)
                pltpu.sync_copy(S_h.at[jv.at[0]], sv)
                pltpu.sync_copy(D_h.at[iv.at[0]], dv)
                bt = btv[...]

                @pl.loop(0, CHUNK)
                def _(r):
                    a = (sv.at[pl.ds(r, 1), pl.ds(0, 16)][...]
                         + dv.at[pl.ds(r, 1), pl.ds(0, 16)][...])
                    al = jnp.maximum(a, NEG * a)
                    ev.at[pl.ds(r, 1), pl.ds(0, 16)][...] = jnp.exp(al - bt)

                pltpu.sync_copy(ev, e_h.at[pl.ds(row * CHUNK, CHUNK)])
                pltpu.sync_copy(ev, sh.at[iv.at[0]], add=True)

        @pl.when(c == 0)
        def _():
            run_edges(jc, ic, Sc, Dc, btc, e_c, shc, EPc)

        @pl.when(c == 1)
        def _():
            run_edges(jw, iw, Sw, Dw, btw, e_w, shw, EPw)
            run_edges(jr, ir, Sr, Dr, btr, e_r, shr, EPr)

        plsc.subcore_barrier()

        def flush(sh, out, nrows):
            rows = nrows // NSUB
            r0 = sub * rows
            pltpu.sync_copy(sh.at[pl.ds(r0, rows)], out.at[pl.ds(r0, rows)])

        @pl.when(c == 0)
        def _():
            flush(shc, s_c, NPP)

        @pl.when(c == 1)
        def _():
            flush(shw, s_w, NPP)
            flush(shr, s_r, NPA)

    return pl.kernel(
        body,
        mesh=_sc_mesh(),
        compiler_params=_sc_params(),
        out_type=[jax.ShapeDtypeStruct((EPc, 16), f32),
                  jax.ShapeDtypeStruct((EPw, 16), f32),
                  jax.ShapeDtypeStruct((EPr, 16), f32),
                  jax.ShapeDtypeStruct((NPP, 16), f32),
                  jax.ShapeDtypeStruct((NPP, 16), f32),
                  jax.ShapeDtypeStruct((NPA, 16), f32)],
        scratch_types=[pltpu.VMEM((1, CHUNK), jnp.int32),
                       pltpu.VMEM((1, CHUNK), jnp.int32),
                       pltpu.VMEM((CHUNK, 16), f32),
                       pltpu.VMEM((CHUNK, 16), f32),
                       pltpu.VMEM((CHUNK, 16), f32),
                       pltpu.VMEM((16,), f32),
                       pltpu.VMEM_SHARED((NPP, 16), f32),
                       pltpu.VMEM_SHARED((NPP, 16), f32),
                       pltpu.VMEM_SHARED((NPA, 16), f32)],
    )


def _sc_aggregate(NPP, NPA, EPc, EPw, EPr):
    f32 = jnp.float32
    SCR = SCHUNK // CHUNK

    def body(jc, ic, jw, iw, jr, ir, ec, ew, er,
             xp0, xp1, xp2, xp3, xa0, xa1, xa2, xa3, z32,
             agg_c, agg_w, agg_r,
             jv0, jv1, iv0, iv1, xv0, xv1, ev, mv,
             is0, is1, gs0, gs1, sh):
        c = lax.axis_index("c")
        sub = lax.axis_index("s")
        xps = (xp0, xp1, xp2, xp3)
        xas = (xa0, xa1, xa2, xa3)
        bufs = ((jv0, iv0, xv0, is0, gs0),
                (jv1, iv1, xv1, is1, gs1))

        def zero_round(nrows):
            rows = nrows // NSUB
            r0 = sub * rows
            pltpu.sync_copy(z32.at[pl.ds(0, rows)], sh.at[pl.ds(r0, rows)])

        def acc_round(j_h, i_h, e_h, xg_h, g, EP):
            nsc = EP // NSUB // SCHUNK
            wr0 = sub * nsc * SCR

            def issue_idx(b, k):
                jb, ib, _, isem, _ = bufs[b]
                br = wr0 + k * SCR
                pltpu.async_copy(j_h.at[pl.ds(br, SCR)], jb, isem)
                pltpu.async_copy(i_h.at[pl.ds(br, SCR)], ib, isem)

            def wait_idx(b):
                jb, ib, _, isem, _ = bufs[b]
                pltpu.make_async_copy(j_h.at[pl.ds(0, SCR)], jb, isem).wait()
                pltpu.make_async_copy(i_h.at[pl.ds(0, SCR)], ib, isem).wait()

            def issue_g(b):
                jb, _, xb, _, gsem = bufs[b]
                for q in range(SCR):
                    pltpu.async_copy(xg_h.at[jb.at[q]],
                                     xb.at[pl.ds(q * CHUNK, CHUNK)], gsem)

            def wait_g(b):
                jb, _, xb, _, gsem = bufs[b]
                for q in range(SCR):
                    pltpu.make_async_copy(
                        xg_h.at[jb.at[q]],
                        xb.at[pl.ds(q * CHUNK, CHUNK)], gsem).wait()

            def compute_scatter(b, k):
                _, ib, xb, _, _ = bufs[b]
                pltpu.sync_copy(
                    e_h.at[pl.ds((wr0 + k * SCR) * CHUNK, SCHUNK)], ev)

                @pl.loop(0, SCHUNK)
                def _(r):
                    er = ev.at[pl.ds(r, 1), pl.ds(0, 16)][...]
                    e0 = er[0, 2 * g]
                    e1 = er[0, 2 * g + 1]
                    x0 = xb.at[pl.ds(r, 1), pl.ds(0, 16)][...]
                    x1 = xb.at[pl.ds(r, 1), pl.ds(16, 16)][...]
                    mv.at[pl.ds(r, 1), pl.ds(0, 16)][...] = x0 * e0
                    mv.at[pl.ds(r, 1), pl.ds(16, 16)][...] = x1 * e1

                for q in range(SCR):
                    pltpu.sync_copy(mv.at[pl.ds(q * CHUNK, CHUNK)],
                                    sh.at[ib.at[q]], add=True)

            issue_idx(0, 0)
            wait_idx(0)
            issue_g(0)
            issue_idx(1, 1)

            @pl.loop(0, nsc, step=2)
            def _(k):
                wait_idx(1)
                issue_g(1)
                wait_g(0)
                compute_scatter(0, k)

                @pl.when(k + 2 < nsc)
                def _():
                    issue_idx(0, k + 2)
                    wait_idx(0)
                    issue_g(0)

                wait_g(1)
                compute_scatter(1, k + 1)

                @pl.when(k + 3 < nsc)
                def _():
                    issue_idx(1, k + 3)

        def flush_round(agg_out, g, nrows):
            rows = nrows // NSUB
            r0 = sub * rows
            pltpu.sync_copy(sh.at[pl.ds(r0, rows)],
                            agg_out.at[g, pl.ds(r0, rows)])

        for r in range(8):
            g = r % 4
            if r < 4:
                @pl.when(c == 0)
                def _():
                    zero_round(NPP)

            @pl.when(c == 1)
            def _():
                zero_round(NPP if r < 4 else NPA)

            plsc.subcore_barrier()

            if r < 4:
                @pl.when(c == 0)
                def _():
                    acc_round(jc, ic, ec, xps[g], g, EPc)

            @pl.when(c == 1)
            def _():
                if r < 4:
                    acc_round(jw, iw, ew, xas[g], g, EPw)
                else:
                    acc_round(jr, ir, er, xps[g], g, EPr)

            plsc.subcore_barrier()

            if r < 4:
                @pl.when(c == 0)
                def _():
                    flush_round(agg_c, g, NPP)

            @pl.when(c == 1)
            def _():
                if r < 4:
                    flush_round(agg_w, g, NPP)
                else:
                    flush_round(agg_r, g, NPA)

            plsc.subcore_barrier()

    return pl.kernel(
        body,
        mesh=_sc_mesh(),
        compiler_params=_sc_params(),
        out_type=[jax.ShapeDtypeStruct((NG, NPP, GW), f32),
                  jax.ShapeDtypeStruct((NG, NPP, GW), f32),
                  jax.ShapeDtypeStruct((NG, NPA, GW), f32)],
        scratch_types=[pltpu.VMEM((SCHUNK // CHUNK, CHUNK), jnp.int32),
                       pltpu.VMEM((SCHUNK // CHUNK, CHUNK), jnp.int32),
                       pltpu.VMEM((SCHUNK // CHUNK, CHUNK), jnp.int32),
                       pltpu.VMEM((SCHUNK // CHUNK, CHUNK), jnp.int32),
                       pltpu.VMEM((SCHUNK, GW), f32),
                       pltpu.VMEM((SCHUNK, GW), f32),
                       pltpu.VMEM((SCHUNK, 16), f32),
                       pltpu.VMEM((SCHUNK, GW), f32),
                       pltpu.SemaphoreType.DMA,
                       pltpu.SemaphoreType.DMA,
                       pltpu.SemaphoreType.DMA,
                       pltpu.SemaphoreType.DMA,
                       pltpu.VMEM_SHARED((NPP, GW), f32)],
    )


def _att_mat(a):
    m = jnp.einsum('hd,hk->hdk', a, jnp.eye(HEADS, dtype=a.dtype))
    return jnp.pad(m.reshape(HID, HEADS), ((0, 0), (0, 8)))


def _bound(Ts, Td):
    raw = Ts.max(0) + Td.max(0)
    return jnp.maximum(raw, NEG * raw)


def kernel(x_paper, x_author, ei_cites, ei_writes, ei_rev, params):
    NPv, NAv = x_paper.shape[0], x_author.shape[0]
    NPP = _rup(NPv + 1, BN)
    NPA = _rup(NAv + 1, BN)
    Ec, Ew, Er = ei_cites.shape[1], ei_writes.shape[1], ei_rev.shape[1]
    EPc, EPw, EPr = _rup(Ec, EALIGN), _rup(Ew, EALIGN), _rup(Er, EALIGN)

    xp = jnp.pad(x_paper, ((0, NPP - NPv), (0, 0)))
    xa = jnp.pad(x_author, ((0, NPA - NAv), (0, 0)))

    def pad_ei(ei, EP, dsrc, ddst):
        j = jnp.pad(ei[0], (0, EP - ei.shape[1]), constant_values=dsrc)
        i = jnp.pad(ei[1], (0, EP - ei.shape[1]), constant_values=ddst)
        return j.reshape(EP // CHUNK, CHUNK), i.reshape(EP // CHUNK, CHUNK)

    jc, ic = pad_ei(ei_cites, EPc, NPv, NPv)
    jw, iw = pad_ei(ei_writes, EPw, NAv, NPv)
    jr, ir = pad_ei(ei_rev, EPr, NPv, NAv)

    z16 = jnp.zeros((NPP, 16), jnp.float32)
    z32 = jnp.zeros((NPP, GW), jnp.float32)

    kA = _sc_edge_weights(NPP, NPA, EPc, EPw, EPr)
    kB = _sc_aggregate(NPP, NPA, EPc, EPw, EPr)

    state = None
    for li in ('l1', 'l2'):
        lp = params[li]
        Wp, bp = lp['proj']['paper']
        Wa, ba = lp['proj']['author']
        asc, adc = lp['att']['cites']
        asw, adw = lp['att']['writes']
        asr, adr = lp['att']['rev']
        Ap = jnp.concatenate([_att_mat(asc), _att_mat(adc),
                              _att_mat(adw), _att_mat(asr)], axis=1)
        Aa = jnp.concatenate([_att_mat(asw), _att_mat(adr)], axis=1)

        if state is None:
            pp = _pre_raw(xp, Wp, bp, Ap, 4)
            pa = _pre_raw(xa, Wa, ba, Aa, 2)
        else:
            oc, ow, attn, agg_r_prev, s_r_prev = state
            pp = _pre_mix(oc, ow, attn, Wp, bp, Ap, 4)
            pa = _pre_agg(agg_r_prev, s_r_prev, Wa, ba, Aa, 2)
        xg_p = pp[:NG]
        Sc, Dc, Dw, Sr = pp[NG:]
        xg_a = pa[:NG]
        Sw, Dr = pa[NG:]

        btc = _bound(Sc, Dc)
        btw = _bound(Sw, Dw)
        btr = _bound(Sr, Dr)

        e_c, e_w, e_r, s_c, s_w, s_r = kA(
            jc, ic, jw, iw, jr, ir, Sc, Dc, Sw, Dw, Sr, Dr,
            btc, btw, btr, z16)
        agg_c, agg_w, agg_r = kB(
            jc, ic, jw, iw, jr, ir, e_c, e_w, e_r,
            *xg_p, *xg_a, z32)

        kW, kb = lp['k']
        q = lp['q']
        oc, ow, tsum = _post_p(agg_c, s_c, agg_w, s_w, kW, kb, NPv)
        score = ((tsum / NPv) * q[None, :]).sum(-1)
        attn = jax.nn.softmax(score)
        state = (oc, ow, attn, agg_r, s_r)

    oc, ow, attn, agg_r, s_r = state
    W, b = params['lin']
    out_p = _final_mix(oc, ow, attn, W, b)[:NPv]
    out_a = _final_agg(agg_r, s_r, W, b)[:NAv]
    return (out_p, out_a)


# speedup vs baseline: 73.0049x; 1.5998x over previous
import jax
import jax.numpy as jnp
from jax import lax
from jax.experimental import pallas as pl
from jax.experimental.pallas import tpu as pltpu
from jax.experimental.pallas import tpu_sc as plsc

HEADS = 8
HID = 128
GH = 16
NG = 4
GW = 32
NEG = 0.2
NSUB = 16
CHUNK = 128
SCHUNK = 256
EALIGN = NSUB * 512
BN = 1024
EPS = 1e-16


def _rup(n, m):
    return ((n + m - 1) // m) * m


def _proj_tail(y, a_ref, outs, ntabs):
    xg = outs[:NG]
    ts = outs[NG:]
    for g in range(NG):
        xg[g][...] = y[:, g * GW:(g + 1) * GW]
    t = jnp.dot(y, a_ref[...], preferred_element_type=jnp.float32)
    for k in range(ntabs):
        ts[k][...] = t[:, k * 16:(k + 1) * 16]


def _pre_raw_body(ntabs, x_ref, w_ref, b_ref, a_ref, *outs):
    y = jnp.dot(x_ref[...], w_ref[...],
                preferred_element_type=jnp.float32) + b_ref[...]
    _proj_tail(y, a_ref, outs, ntabs)


def _elu(x):
    return jnp.where(x > 0, x, jnp.exp(jnp.minimum(x, 0.0)) - 1.0)


def _assemble(a_refs, s_ref):
    cat = jnp.concatenate([a[0] for a in a_refs], axis=-1)
    s = s_ref[...]
    cols = []
    for h in range(HEADS):
        denom = s[:, h:h + 1] + EPS
        cols.append(cat[:, GH * h:GH * (h + 1)] / denom)
    return jnp.maximum(jnp.concatenate(cols, axis=-1), 0.0)


def _pre_mix_body(ntabs, oc_ref, ow_ref, attn_ref, w_ref, b_ref, a_ref, *outs):
    x = _elu(attn_ref[0] * oc_ref[...] + attn_ref[1] * ow_ref[...])
    y = jnp.dot(x, w_ref[...], preferred_element_type=jnp.float32) + b_ref[...]
    _proj_tail(y, a_ref, outs, ntabs)


def _pre_agg_body(ntabs, a0, a1, a2, a3, s_ref, w_ref, b_ref, a_ref, *outs):
    x = _elu(_assemble((a0, a1, a2, a3), s_ref))
    y = jnp.dot(x, w_ref[...], preferred_element_type=jnp.float32) + b_ref[...]
    _proj_tail(y, a_ref, outs, ntabs)


def _post_p_body(nvalid, c0, c1, c2, c3, sc_ref, w0, w1, w2, w3, sw_ref,
                 kw_ref, kb_ref, oc_ref, ow_ref, ts_ref):
    i = pl.program_id(0)
    oc = _assemble((c0, c1, c2, c3), sc_ref)
    ow = _assemble((w0, w1, w2, w3), sw_ref)
    oc_ref[...] = oc
    ow_ref[...] = ow
    tc = jnp.tanh(jnp.dot(oc, kw_ref[...],
                          preferred_element_type=jnp.float32) + kb_ref[...])
    tw = jnp.tanh(jnp.dot(ow, kw_ref[...],
                          preferred_element_type=jnp.float32) + kb_ref[...])
    ridx = i * BN + lax.broadcasted_iota(jnp.int32, (BN, 1), 0)
    m = (ridx < nvalid).astype(jnp.float32)
    part = jnp.stack([(tc * m).sum(0), (tw * m).sum(0)])

    @pl.when(i == 0)
    def _():
        ts_ref[...] = part

    @pl.when(i > 0)
    def _():
        ts_ref[...] = ts_ref[...] + part


def _final_mix_body(oc_ref, ow_ref, attn_ref, w_ref, b_ref, o_ref):
    x = _elu(attn_ref[0] * oc_ref[...] + attn_ref[1] * ow_ref[...])
    o_ref[...] = jnp.dot(x, w_ref[...],
                         preferred_element_type=jnp.float32) + b_ref[...]


def _final_agg_body(a0, a1, a2, a3, s_ref, w_ref, b_ref, o_ref):
    x = _elu(_assemble((a0, a1, a2, a3), s_ref))
    o_ref[...] = jnp.dot(x, w_ref[...],
                         preferred_element_type=jnp.float32) + b_ref[...]


def _mat_spec(shape):
    nd = len(shape)
    return pl.BlockSpec(shape, lambda i, _nd=nd: (0,) * _nd)


def _agg_view_specs():
    return [pl.BlockSpec((1, BN, GW), lambda i, g=g: (g, i, 0))
            for g in range(NG)]


def _pre_outs(npad, ntabs):
    shapes = ([jax.ShapeDtypeStruct((npad, GW), jnp.float32)] * NG
              + [jax.ShapeDtypeStruct((npad, 16), jnp.float32)] * ntabs)
    specs = ([pl.BlockSpec((BN, GW), lambda i: (i, 0))] * NG
             + [pl.BlockSpec((BN, 16), lambda i: (i, 0))] * ntabs)
    return shapes, specs


def _pre_raw(x, W, b, A, ntabs):
    npad = x.shape[0]
    oshapes, ospecs = _pre_outs(npad, ntabs)
    import functools
    return pl.pallas_call(
        functools.partial(_pre_raw_body, ntabs),
        grid=(npad // BN,),
        in_specs=[pl.BlockSpec((BN, HID), lambda i: (i, 0)),
                  _mat_spec((HID, HID)), _mat_spec((1, HID)),
                  _mat_spec((HID, 16 * ntabs))],
        out_specs=ospecs, out_shape=oshapes,
    )(x, W, b.reshape(1, HID), A)


def _pre_mix(oc, ow, attn, W, b, A, ntabs):
    npad = oc.shape[0]
    oshapes, ospecs = _pre_outs(npad, ntabs)
    import functools
    return pl.pallas_call(
        functools.partial(_pre_mix_body, ntabs),
        grid=(npad // BN,),
        in_specs=[pl.BlockSpec((BN, HID), lambda i: (i, 0)),
                  pl.BlockSpec((BN, HID), lambda i: (i, 0)),
                  pl.BlockSpec(memory_space=pltpu.SMEM),
                  _mat_spec((HID, HID)), _mat_spec((1, HID)),
                  _mat_spec((HID, 16 * ntabs))],
        out_specs=ospecs, out_shape=oshapes,
    )(oc, ow, attn, W, b.reshape(1, HID), A)


def _pre_agg(agg, s, W, b, A, ntabs):
    npad = s.shape[0]
    oshapes, ospecs = _pre_outs(npad, ntabs)
    import functools
    return pl.pallas_call(
        functools.partial(_pre_agg_body, ntabs),
        grid=(npad // BN,),
        in_specs=_agg_view_specs()
        + [pl.BlockSpec((BN, 16), lambda i: (i, 0)),
           _mat_spec((HID, HID)), _mat_spec((1, HID)),
           _mat_spec((HID, 16 * ntabs))],
        out_specs=ospecs, out_shape=oshapes,
    )(agg, agg, agg, agg, s, W, b.reshape(1, HID), A)


def _post_p(agg_c, s_c, agg_w, s_w, kW, kb, nvalid):
    npad = s_c.shape[0]
    import functools
    return pl.pallas_call(
        functools.partial(_post_p_body, nvalid),
        grid=(npad // BN,),
        in_specs=_agg_view_specs()
        + [pl.BlockSpec((BN, 16), lambda i: (i, 0))]
        + _agg_view_specs()
        + [pl.BlockSpec((BN, 16), lambda i: (i, 0)),
           _mat_spec((HID, HID)), _mat_spec((1, HID))],
        out_specs=[pl.BlockSpec((BN, HID), lambda i: (i, 0)),
                   pl.BlockSpec((BN, HID), lambda i: (i, 0)),
                   pl.BlockSpec((2, HID), lambda i: (0, 0))],
        out_shape=[jax.ShapeDtypeStruct((npad, HID), jnp.float32),
                   jax.ShapeDtypeStruct((npad, HID), jnp.float32),
                   jax.ShapeDtypeStruct((2, HID), jnp.float32)],
    )(agg_c, agg_c, agg_c, agg_c, s_c, agg_w, agg_w, agg_w, agg_w, s_w,
      kW, kb.reshape(1, HID))


def _final_mix(oc, ow, attn, W, b):
    npad = oc.shape[0]
    return pl.pallas_call(
        _final_mix_body,
        grid=(npad // BN,),
        in_specs=[pl.BlockSpec((BN, HID), lambda i: (i, 0)),
                  pl.BlockSpec((BN, HID), lambda i: (i, 0)),
                  pl.BlockSpec(memory_space=pltpu.SMEM),
                  _mat_spec((HID, HID)), _mat_spec((1, HID))],
        out_specs=pl.BlockSpec((BN, HID), lambda i: (i, 0)),
        out_shape=jax.ShapeDtypeStruct((npad, HID), jnp.float32),
    )(oc, ow, attn, W, b.reshape(1, HID))


def _final_agg(agg, s, W, b):
    npad = s.shape[0]
    return pl.pallas_call(
        _final_agg_body,
        grid=(npad // BN,),
        in_specs=_agg_view_specs()
        + [pl.BlockSpec((BN, 16), lambda i: (i, 0)),
           _mat_spec((HID, HID)), _mat_spec((1, HID))],
        out_specs=pl.BlockSpec((BN, HID), lambda i: (i, 0)),
        out_shape=jax.ShapeDtypeStruct((npad, HID), jnp.float32),
    )(agg, agg, agg, agg, s, W, b.reshape(1, HID))


def _sc_mesh():
    return plsc.VectorSubcoreMesh(core_axis_name="c", subcore_axis_name="s")


def _sc_params():
    import dataclasses
    cp = pltpu.CompilerParams(use_tc_tiling_on_sc=False)
    if "needs_layout_passes" in pltpu.CompilerParams.__dataclass_fields__:
        cp = dataclasses.replace(cp, needs_layout_passes=False)
    return cp


def _sc_edge_weights(NPP, NPA, EPc, EPw, EPr):
    f32 = jnp.float32
    SCA = 512
    SCRA = SCA // CHUNK
    PK = SCA // 8

    def body(jc, ic, jw, iw, jr, ir, Sc, Dc, Sw, Dw, Sr, Dr,
             btc, btw, btr, z16,
             e_c, e_w, e_r, s_c, s_w, s_r,
             jv0, jv1, iv0, iv1, sv0, sv1, dv0, dv1, ev, egb, btv,
             is0, is1, gs0, gs1, shA, shB):
        c = lax.axis_index("c")
        sub = lax.axis_index("s")
        bufs = ((jv0, iv0, sv0, dv0, is0, gs0),
                (jv1, iv1, sv1, dv1, is1, gs1))
        lane = lax.iota(jnp.int32, 16)
        rhalf = lax.shift_right_logical(lane, 1)
        lbit = lax.bitwise_and(lane, 1)

        def zero_tab(sh, nrows):
            rows = nrows // NSUB
            r0 = sub * rows
            pltpu.sync_copy(z16.at[pl.ds(0, rows)], sh.at[pl.ds(r0, rows)])

        @pl.when(c == 0)
        def _():
            zero_tab(shA, NPP)

        @pl.when(c == 1)
        def _():
            zero_tab(shA, NPP)
            zero_tab(shB, NPA)

        plsc.subcore_barrier()

        def run_edges(j_h, i_h, S_h, D_h, bt_h, e_h, sh, EP):
            pltpu.sync_copy(bt_h, btv)
            nsc = EP // NSUB // SCA
            wr0 = sub * nsc * SCRA

            def issue_idx(b, k):
                jb, ib, _, _, isem, _ = bufs[b]
                br = wr0 + k * SCRA
                pltpu.async_copy(j_h.at[pl.ds(br, SCRA)], jb, isem)
                pltpu.async_copy(i_h.at[pl.ds(br, SCRA)], ib, isem)

            def wait_idx(b):
                jb, ib, _, _, isem, _ = bufs[b]
                pltpu.make_async_copy(j_h.at[pl.ds(0, SCRA)], jb, isem).wait()
                pltpu.make_async_copy(i_h.at[pl.ds(0, SCRA)], ib, isem).wait()

            def issue_g(b):
                jb, ib, sb, db, _, gsem = bufs[b]
                for q in range(SCRA):
                    pltpu.async_copy(S_h.at[jb.at[q]],
                                     sb.at[pl.ds(q * CHUNK, CHUNK)], gsem)
                    pltpu.async_copy(D_h.at[ib.at[q]],
                                     db.at[pl.ds(q * CHUNK, CHUNK)], gsem)

            def wait_g(b):
                jb, ib, sb, db, _, gsem = bufs[b]
                for q in range(SCRA):
                    pltpu.make_async_copy(
                        S_h.at[jb.at[q]],
                        sb.at[pl.ds(q * CHUNK, CHUNK)], gsem).wait()
                    pltpu.make_async_copy(
                        D_h.at[ib.at[q]],
                        db.at[pl.ds(q * CHUNK, CHUNK)], gsem).wait()

            def compute_scatter(b, k):
                _, ib, sb, db, _, _ = bufs[b]
                bt = btv[...]

                @pl.loop(0, SCA)
                def _(r):
                    a = sb[r] + db[r]
                    al = jnp.maximum(a, NEG * a)
                    ev[r] = jnp.exp(al - bt)

                for g in range(NG):
                    @pl.loop(0, PK)
                    def _(k8, _g=g):
                        v = plsc.load_gather(
                            ev, [k8 * 8 + rhalf, 2 * _g + lbit])
                        egb[PK * _g + k8] = v

                prow = (wr0 + k * SCRA) * 16
                for g in range(NG):
                    pltpu.sync_copy(egb.at[pl.ds(PK * g, PK)],
                                    e_h.at[g, pl.ds(prow, PK)])
                for q in range(SCRA):
                    pltpu.sync_copy(ev.at[pl.ds(q * CHUNK, CHUNK)],
                                    sh.at[ib.at[q]], add=True)

            issue_idx(0, 0)
            wait_idx(0)
            issue_g(0)
            issue_idx(1, 1)

            @pl.loop(0, nsc, step=2)
            def _(k):
                wait_idx(1)
                issue_g(1)
                wait_g(0)
                compute_scatter(0, k)

                @pl.when(k + 2 < nsc)
                def _():
                    issue_idx(0, k + 2)
                    wait_idx(0)
                    issue_g(0)

                wait_g(1)
                compute_scatter(1, k + 1)

                @pl.when(k + 3 < nsc)
                def _():
                    issue_idx(1, k + 3)

        @pl.when(c == 0)
        def _():
            run_edges(jc, ic, Sc, Dc, btc, e_c, shA, EPc)

        @pl.when(c == 1)
        def _():
            run_edges(jw, iw, Sw, Dw, btw, e_w, shA, EPw)
            run_edges(jr, ir, Sr, Dr, btr, e_r, shB, EPr)

        plsc.subcore_barrier()

        def flush(sh, out, nrows):
            rows = nrows // NSUB
            r0 = sub * rows
            pltpu.sync_copy(sh.at[pl.ds(r0, rows)], out.at[pl.ds(r0, rows)])

        @pl.when(c == 0)
        def _():
            flush(shA, s_c, NPP)

        @pl.when(c == 1)
        def _():
            flush(shA, s_w, NPP)
            flush(shB, s_r, NPA)

    return pl.kernel(
        body,
        mesh=_sc_mesh(),
        compiler_params=_sc_params(),
        out_type=[jax.ShapeDtypeStruct((NG, EPc // 8, 16), f32),
                  jax.ShapeDtypeStruct((NG, EPw // 8, 16), f32),
                  jax.ShapeDtypeStruct((NG, EPr // 8, 16), f32),
                  jax.ShapeDtypeStruct((NPP, 16), f32),
                  jax.ShapeDtypeStruct((NPP, 16), f32),
                  jax.ShapeDtypeStruct((NPA, 16), f32)],
        scratch_types=[pltpu.VMEM((SCRA, CHUNK), jnp.int32),
                       pltpu.VMEM((SCRA, CHUNK), jnp.int32),
                       pltpu.VMEM((SCRA, CHUNK), jnp.int32),
                       pltpu.VMEM((SCRA, CHUNK), jnp.int32),
                       pltpu.VMEM((SCA, 16), f32),
                       pltpu.VMEM((SCA, 16), f32),
                       pltpu.VMEM((SCA, 16), f32),
                       pltpu.VMEM((SCA, 16), f32),
                       pltpu.VMEM((SCA, 16), f32),
                       pltpu.VMEM((NG * PK, 16), f32),
                       pltpu.VMEM((16,), f32),
                       pltpu.SemaphoreType.DMA,
                       pltpu.SemaphoreType.DMA,
                       pltpu.SemaphoreType.DMA,
                       pltpu.SemaphoreType.DMA,
                       pltpu.VMEM_SHARED((NPP, 16), f32),
                       pltpu.VMEM_SHARED((NPA, 16), f32)],
    )


def _sc_aggregate(NPP, NPA, EPc, EPw, EPr):
    f32 = jnp.float32
    SCR = SCHUNK // CHUNK

    def body(jc, ic, jw, iw, jr, ir, ec, ew, er,
             xp0, xp1, xp2, xp3, xa0, xa1, xa2, xa3, z32,
             agg_c, agg_w, agg_r,
             jv0, jv1, iv0, iv1, xv0, xv1, eb0, eb1, mv,
             is0, is1, gs0, gs1, sh):
        c = lax.axis_index("c")
        sub = lax.axis_index("s")
        xps = (xp0, xp1, xp2, xp3)
        xas = (xa0, xa1, xa2, xa3)
        bufs = ((jv0, iv0, xv0, eb0, is0, gs0),
                (jv1, iv1, xv1, eb1, is1, gs1))

        def zero_round(nrows):
            rows = nrows // NSUB
            r0 = sub * rows
            pltpu.sync_copy(z32.at[pl.ds(0, rows)], sh.at[pl.ds(r0, rows)])

        def acc_round(j_h, i_h, e_h, xg_h, g, EP):
            nsc = EP // NSUB // SCHUNK
            wr0 = sub * nsc * SCR
            PKB = SCR * 16

            def issue_idx(b, k):
                jb, ib, _, eb, isem, _ = bufs[b]
                br = wr0 + k * SCR
                pltpu.async_copy(j_h.at[pl.ds(br, SCR)], jb, isem)
                pltpu.async_copy(i_h.at[pl.ds(br, SCR)], ib, isem)
                pltpu.async_copy(e_h.at[g, pl.ds(br * 16, PKB)], eb, isem)

            def wait_idx(b):
                jb, ib, _, eb, isem, _ = bufs[b]
                pltpu.make_async_copy(j_h.at[pl.ds(0, SCR)], jb, isem).wait()
                pltpu.make_async_copy(i_h.at[pl.ds(0, SCR)], ib, isem).wait()
                pltpu.make_async_copy(e_h.at[g, pl.ds(0, PKB)], eb,
                                      isem).wait()

            def issue_g(b):
                jb, _, xb, _, _, gsem = bufs[b]
                for q in range(SCR):
                    pltpu.async_copy(xg_h.at[jb.at[q]],
                                     xb.at[pl.ds(q * CHUNK, CHUNK)], gsem)

            def wait_g(b):
                jb, _, xb, _, _, gsem = bufs[b]
                for q in range(SCR):
                    pltpu.make_async_copy(
                        xg_h.at[jb.at[q]],
                        xb.at[pl.ds(q * CHUNK, CHUNK)], gsem).wait()

            def compute_scatter(b, k):
                _, ib, xb, eb, _, _ = bufs[b]

                @pl.loop(0, PKB)
                def _(k8):
                    er = eb[k8]
                    for t in range(8):
                        r = k8 * 8 + t
                        e0 = er[2 * t]
                        e1 = er[2 * t + 1]
                        x0 = xb.at[r, pl.ds(0, 16)][...]
                        x1 = xb.at[r, pl.ds(16, 16)][...]
                        mv.at[r, pl.ds(0, 16)][...] = x0 * e0
                        mv.at[r, pl.ds(16, 16)][...] = x1 * e1

                for q in range(SCR):
                    pltpu.sync_copy(mv.at[pl.ds(q * CHUNK, CHUNK)],
                                    sh.at[ib.at[q]], add=True)

            issue_idx(0, 0)
            wait_idx(0)
            issue_g(0)
            issue_idx(1, 1)

            @pl.loop(0, nsc, step=2)
            def _(k):
                wait_idx(1)
                issue_g(1)
                wait_g(0)
                compute_scatter(0, k)

                @pl.when(k + 2 < nsc)
                def _():
                    issue_idx(0, k + 2)
                    wait_idx(0)
                    issue_g(0)

                wait_g(1)
                compute_scatter(1, k + 1)

                @pl.when(k + 3 < nsc)
                def _():
                    issue_idx(1, k + 3)

        def flush_round(agg_out, g, nrows):
            rows = nrows // NSUB
            r0 = sub * rows
            pltpu.sync_copy(sh.at[pl.ds(r0, rows)],
                            agg_out.at[g, pl.ds(r0, rows)])

        for r in range(8):
            g = r % 4
            if r < 4:
                @pl.when(c == 0)
                def _():
                    zero_round(NPP)

            @pl.when(c == 1)
            def _():
                zero_round(NPP if r < 4 else NPA)

            plsc.subcore_barrier()

            if r < 4:
                @pl.when(c == 0)
                def _():
                    acc_round(jc, ic, ec, xps[g], g, EPc)

            @pl.when(c == 1)
            def _():
                if r < 4:
                    acc_round(jw, iw, ew, xas[g], g, EPw)
                else:
                    acc_round(jr, ir, er, xps[g], g, EPr)

            plsc.subcore_barrier()

            if r < 4:
                @pl.when(c == 0)
                def _():
                    flush_round(agg_c, g, NPP)

            @pl.when(c == 1)
            def _():
                if r < 4:
                    flush_round(agg_w, g, NPP)
                else:
                    flush_round(agg_r, g, NPA)

            plsc.subcore_barrier()

    return pl.kernel(
        body,
        mesh=_sc_mesh(),
        compiler_params=_sc_params(),
        out_type=[jax.ShapeDtypeStruct((NG, NPP, GW), f32),
                  jax.ShapeDtypeStruct((NG, NPP, GW), f32),
                  jax.ShapeDtypeStruct((NG, NPA, GW), f32)],
        scratch_types=[pltpu.VMEM((SCHUNK // CHUNK, CHUNK), jnp.int32),
                       pltpu.VMEM((SCHUNK // CHUNK, CHUNK), jnp.int32),
                       pltpu.VMEM((SCHUNK // CHUNK, CHUNK), jnp.int32),
                       pltpu.VMEM((SCHUNK // CHUNK, CHUNK), jnp.int32),
                       pltpu.VMEM((SCHUNK, GW), f32),
                       pltpu.VMEM((SCHUNK, GW), f32),
                       pltpu.VMEM((SCHUNK // 8, 16), f32),
                       pltpu.VMEM((SCHUNK // 8, 16), f32),
                       pltpu.VMEM((SCHUNK, GW), f32),
                       pltpu.SemaphoreType.DMA,
                       pltpu.SemaphoreType.DMA,
                       pltpu.SemaphoreType.DMA,
                       pltpu.SemaphoreType.DMA,
                       pltpu.VMEM_SHARED((NPP, GW), f32)],
    )


def _att_mat(a):
    m = jnp.einsum('hd,hk->hdk', a, jnp.eye(HEADS, dtype=a.dtype))
    return jnp.pad(m.reshape(HID, HEADS), ((0, 0), (0, 8)))


def _bound(Ts, Td):
    raw = Ts.max(0) + Td.max(0)
    return jnp.maximum(raw, NEG * raw)


def kernel(x_paper, x_author, ei_cites, ei_writes, ei_rev, params):
    NPv, NAv = x_paper.shape[0], x_author.shape[0]
    NPP = _rup(NPv + 1, BN)
    NPA = _rup(NAv + 1, BN)
    Ec, Ew, Er = ei_cites.shape[1], ei_writes.shape[1], ei_rev.shape[1]
    EPc, EPw, EPr = _rup(Ec, EALIGN), _rup(Ew, EALIGN), _rup(Er, EALIGN)

    xp = jnp.pad(x_paper, ((0, NPP - NPv), (0, 0)))
    xa = jnp.pad(x_author, ((0, NPA - NAv), (0, 0)))

    def pad_ei(ei, EP, dsrc, ddst):
        j = jnp.pad(ei[0], (0, EP - ei.shape[1]), constant_values=dsrc)
        i = jnp.pad(ei[1], (0, EP - ei.shape[1]), constant_values=ddst)
        return j.reshape(EP // CHUNK, CHUNK), i.reshape(EP // CHUNK, CHUNK)

    jc, ic = pad_ei(ei_cites, EPc, NPv, NPv)
    jw, iw = pad_ei(ei_writes, EPw, NAv, NPv)
    jr, ir = pad_ei(ei_rev, EPr, NPv, NAv)

    z16 = jnp.zeros((NPP, 16), jnp.float32)
    z32 = jnp.zeros((NPP, GW), jnp.float32)

    kA = _sc_edge_weights(NPP, NPA, EPc, EPw, EPr)
    kB = _sc_aggregate(NPP, NPA, EPc, EPw, EPr)

    state = None
    for li in ('l1', 'l2'):
        lp = params[li]
        Wp, bp = lp['proj']['paper']
        Wa, ba = lp['proj']['author']
        asc, adc = lp['att']['cites']
        asw, adw = lp['att']['writes']
        asr, adr = lp['att']['rev']
        Ap = jnp.concatenate([_att_mat(asc), _att_mat(adc),
                              _att_mat(adw), _att_mat(asr)], axis=1)
        Aa = jnp.concatenate([_att_mat(asw), _att_mat(adr)], axis=1)

        if state is None:
            pp = _pre_raw(xp, Wp, bp, Ap, 4)
            pa = _pre_raw(xa, Wa, ba, Aa, 2)
        else:
            oc, ow, attn, agg_r_prev, s_r_prev = state
            pp = _pre_mix(oc, ow, attn, Wp, bp, Ap, 4)
            pa = _pre_agg(agg_r_prev, s_r_prev, Wa, ba, Aa, 2)
        xg_p = pp[:NG]
        Sc, Dc, Dw, Sr = pp[NG:]
        xg_a = pa[:NG]
        Sw, Dr = pa[NG:]

        btc = _bound(Sc, Dc)
        btw = _bound(Sw, Dw)
        btr = _bound(Sr, Dr)

        e_c, e_w, e_r, s_c, s_w, s_r = kA(
            jc, ic, jw, iw, jr, ir, Sc, Dc, Sw, Dw, Sr, Dr,
            btc, btw, btr, z16)
        agg_c, agg_w, agg_r = kB(
            jc, ic, jw, iw, jr, ir, e_c, e_w, e_r,
            *xg_p, *xg_a, z32)

        kW, kb = lp['k']
        q = lp['q']
        oc, ow, tsum = _post_p(agg_c, s_c, agg_w, s_w, kW, kb, NPv)
        score = ((tsum / NPv) * q[None, :]).sum(-1)
        attn = jax.nn.softmax(score)
        state = (oc, ow, attn, agg_r, s_r)

    oc, ow, attn, agg_r, s_r = state
    W, b = params['lin']
    out_p = _final_mix(oc, ow, attn, W, b)[:NPv]
    out_a = _final_agg(agg_r, s_r, W, b)[:NAv]
    return (out_p, out_a)
